# baseline (speedup 1.0000x reference)
import functools

import jax
import jax.numpy as jnp
from jax import lax
from jax.experimental import pallas as pl
from jax.experimental.pallas import tpu as pltpu
from jax.experimental.pallas import tpu_sc as plsc

VOCAB = 1000000
D_MODEL = 64
SEQ_LEN = 200
BATCH = 1024

NUM_CORES = 2
NUM_SUBCORES = 16
NW = NUM_CORES * NUM_SUBCORES

ROWS_TOTAL = BATCH * SEQ_LEN
ROWS_PER_W = ROWS_TOTAL // NW
CHUNK = 100
CHUNKS_PER_W = ROWS_PER_W // CHUNK
HALF = SEQ_LEN // CHUNK


def _body(x_hbm, table_hbm, pos_hbm, out_hbm, idx_v, pos_v, buf_v, sem):
    wid = lax.axis_index("s") * NUM_CORES + lax.axis_index("c")

    pltpu.sync_copy(x_hbm.at[wid], idx_v)
    pltpu.sync_copy(pos_hbm, pos_v)

    def chunk_pair(c2, carry):
        for par in range(HALF):
            c = c2 * HALF + par
            pltpu.async_copy(table_hbm.at[idx_v.at[c]], buf_v.at[par], sem).wait()

            def add_row(r, carry2):
                for cc in range(D_MODEL // 16):
                    sl = pl.ds(cc * 16, 16)
                    buf_v[par, r, sl] = buf_v[par, r, sl] + pos_v[par, r, sl]
                return carry2

            lax.fori_loop(0, CHUNK, add_row, 0)
            pltpu.sync_copy(buf_v.at[par], out_hbm.at[wid, c])
        return carry

    lax.fori_loop(0, CHUNKS_PER_W // HALF, chunk_pair, 0)


@jax.jit
def _run(x_r, table, pos_r):
    mesh = plsc.VectorSubcoreMesh(core_axis_name="c", subcore_axis_name="s")
    k = functools.partial(
        pl.kernel,
        mesh=mesh,
        out_type=jax.ShapeDtypeStruct((NW, CHUNKS_PER_W, CHUNK, D_MODEL), jnp.float32),
        scratch_types=[
            pltpu.VMEM((CHUNKS_PER_W, CHUNK), jnp.int32),
            pltpu.VMEM((HALF, CHUNK, D_MODEL), jnp.float32),
            pltpu.VMEM((HALF, CHUNK, D_MODEL), jnp.float32),
            pltpu.SemaphoreType.DMA,
        ],
        compiler_params=pltpu.CompilerParams(use_tc_tiling_on_sc=False),
    )(_body)
    return k(x_r, table, pos_r)


def kernel(x, table, pos_table):
    x_r = x.reshape(NW, CHUNKS_PER_W, CHUNK)
    pos_r = pos_table.reshape(HALF, CHUNK, D_MODEL)
    out = _run(x_r, table, pos_r)
    return out.reshape(BATCH, SEQ_LEN, D_MODEL)


# speedup vs baseline: 1.0944x; 1.0944x over previous
import functools

import jax
import jax.numpy as jnp
from jax import lax
from jax.experimental import pallas as pl
from jax.experimental.pallas import tpu as pltpu
from jax.experimental.pallas import tpu_sc as plsc

VOCAB = 1000000
D_MODEL = 64
SEQ_LEN = 200
BATCH = 1024

NUM_CORES = 2
NUM_SUBCORES = 16
NW = NUM_CORES * NUM_SUBCORES

ROWS_TOTAL = BATCH * SEQ_LEN
ROWS_PER_W = ROWS_TOTAL // NW
CHUNK = 100
CHUNKS_PER_W = ROWS_PER_W // CHUNK
HALF = SEQ_LEN // CHUNK
NBUF = 4
ROUNDS = CHUNKS_PER_W // NBUF


def _body(x_hbm, table_hbm, pos_hbm, out_hbm,
          idx_v, pos_v, gbuf, wbuf, gsems, wsems):
    wid = lax.axis_index("s") * NUM_CORES + lax.axis_index("c")

    pltpu.sync_copy(x_hbm.at[wid], idx_v)
    pltpu.sync_copy(pos_hbm, pos_v)

    def gather_start(c, b):
        pltpu.make_async_copy(
            table_hbm.at[idx_v.at[c]], gbuf.at[b], gsems.at[b]).start()

    def gather_wait(c, b):
        pltpu.make_async_copy(
            table_hbm.at[idx_v.at[c]], gbuf.at[b], gsems.at[b]).wait()

    def write_start(c, b):
        pltpu.make_async_copy(
            wbuf.at[b], out_hbm.at[wid, c], wsems.at[b]).start()

    def write_wait(c, b):
        pltpu.make_async_copy(
            wbuf.at[b], out_hbm.at[wid, c], wsems.at[b]).wait()

    for b in range(NBUF):
        gather_start(b, b)

    def round_body(r, carry):
        for b in range(NBUF):
            c = r * NBUF + b
            gather_wait(c, b)

            @pl.when(r > 0)
            def _():
                write_wait(c - NBUF, b)

            par = b % HALF

            def add_row(row, carry2):
                for cc in range(D_MODEL // 16):
                    sl = pl.ds(cc * 16, 16)
                    wbuf[b, row, sl] = gbuf[b, row, sl] + pos_v[par, row, sl]
                return carry2

            lax.fori_loop(0, CHUNK, add_row, 0)

            @pl.when(r < ROUNDS - 1)
            def _():
                gather_start(c + NBUF, b)

            write_start(c, b)
        return carry

    lax.fori_loop(0, ROUNDS, round_body, 0)

    for b in range(NBUF):
        write_wait((ROUNDS - 1) * NBUF + b, b)


@jax.jit
def _run(x_r, table, pos_r):
    mesh = plsc.VectorSubcoreMesh(core_axis_name="c", subcore_axis_name="s")
    k = functools.partial(
        pl.kernel,
        mesh=mesh,
        out_type=jax.ShapeDtypeStruct((NW, CHUNKS_PER_W, CHUNK, D_MODEL), jnp.float32),
        scratch_types=[
            pltpu.VMEM((CHUNKS_PER_W, CHUNK), jnp.int32),
            pltpu.VMEM((HALF, CHUNK, D_MODEL), jnp.float32),
            pltpu.VMEM((NBUF, CHUNK, D_MODEL), jnp.float32),
            pltpu.VMEM((NBUF, CHUNK, D_MODEL), jnp.float32),
            pltpu.SemaphoreType.DMA((NBUF,)),
            pltpu.SemaphoreType.DMA((NBUF,)),
        ],
        compiler_params=pltpu.CompilerParams(use_tc_tiling_on_sc=False),
    )(_body)
    return k(x_r, table, pos_r)


def kernel(x, table, pos_table):
    x_r = x.reshape(NW, CHUNKS_PER_W, CHUNK)
    pos_r = pos_table.reshape(HALF, CHUNK, D_MODEL)
    out = _run(x_r, table, pos_r)
    return out.reshape(BATCH, SEQ_LEN, D_MODEL)
